# baseline (speedup 1.0000x reference)
import functools

import jax
import jax.numpy as jnp
from jax import lax
from jax.experimental import pallas as pl
from jax.experimental.pallas import tpu as pltpu
from jax.experimental.pallas import tpu_sc as plsc

_B = 16384
_D = 32
_CHUNK = 128


@functools.cache
def _build(B, D):
    info = plsc.get_sparse_core_info()
    NC, NS, L = info.num_cores, info.num_subcores, info.num_lanes
    NW = NC * NS
    b_per_w = B // NW
    n_chunks = b_per_w // _CHUNK
    n_tiles = b_per_w // L
    mesh = plsc.VectorSubcoreMesh(core_axis_name="c", subcore_axis_name="s")

    @functools.partial(
        pl.kernel,
        mesh=mesh,
        out_type=jax.ShapeDtypeStruct((B,), jnp.float32),
        compiler_params=pltpu.CompilerParams(
            needs_layout_passes=False, use_tc_tiling_on_sc=False
        ),
        scratch_types=[
            pltpu.VMEM((n_chunks, _CHUNK), jnp.int32),
            pltpu.VMEM((n_chunks, _CHUNK), jnp.int32),
            pltpu.VMEM((b_per_w, D), jnp.float32),
            pltpu.VMEM((b_per_w, D), jnp.float32),
            pltpu.VMEM((b_per_w,), jnp.float32),
            pltpu.SemaphoreType.DMA,
        ],
    )
    def k(users_hbm, movies_hbm, uf_hbm, mf_hbm, out_hbm,
          uidx, midx, urows, mrows, outv, sem):
        wid = lax.axis_index("s") * NC + lax.axis_index("c")
        base = wid * b_per_w

        pltpu.sync_copy(users_hbm.at[pl.ds(wid * n_chunks, n_chunks)], uidx)
        pltpu.sync_copy(movies_hbm.at[pl.ds(wid * n_chunks, n_chunks)], midx)

        handles = []
        for j in range(n_chunks):
            dst = urows.at[pl.ds(j * _CHUNK, _CHUNK)]
            handles.append(pltpu.async_copy(uf_hbm.at[uidx.at[j]], dst, sem))
            dst = mrows.at[pl.ds(j * _CHUNK, _CHUNK)]
            handles.append(pltpu.async_copy(mf_hbm.at[midx.at[j]], dst, sem))
        for h in handles:
            h.wait()

        lane = lax.iota(jnp.int32, L)

        def tile_body(t, _):
            rows = t * L + lane
            acc = jnp.zeros((L,), jnp.float32)
            for f in range(D):
                col = jnp.full((L,), f, jnp.int32)
                uf = plsc.load_gather(urows, [rows, col])
                mf = plsc.load_gather(mrows, [rows, col])
                acc = acc + uf * mf
            outv[pl.ds(pl.multiple_of(t * L, L), L)] = acc
            return 0

        lax.fori_loop(0, n_tiles, tile_body, 0)

        pltpu.sync_copy(outv, out_hbm.at[pl.ds(base, b_per_w)])

    return k


def kernel(data, user_factors, movie_factors):
    users = data[:, 0].astype(jnp.int32).reshape(_B // _CHUNK, _CHUNK)
    movies = data[:, 1].astype(jnp.int32).reshape(_B // _CHUNK, _CHUNK)
    return _build(_B, _D)(users, movies, user_factors, movie_factors)
